# Initial kernel scaffold; baseline (speedup 1.0000x reference)
import jax
import jax.numpy as jnp
from jax.experimental import pallas as pl


def kernel(x, edge_index, W1, b1, W2, b2, Wfc, bfc):
    raise NotImplementedError("write your pallas kernel here")


# speedup vs baseline: 6.4216x; 6.4216x over previous
import jax
import jax.numpy as jnp
from jax import lax
from jax.experimental import pallas as pl
from jax.experimental.pallas import tpu as pltpu
from jax.experimental.pallas import tpu_sc as plsc

N = 10000
E = 160000
D_IN = 256
D_H1 = 256
D_H2 = 128
D_OUT = 64

NPAD = 10240
DUMMY = NPAD
ACC_ROWS = 10368
CHUNK = 128
NCHUNK = 80
EPAD = 16 * NCHUNK * CHUNK
R_BLK = 1024

_mesh = plsc.VectorSubcoreMesh(core_axis_name="c", subcore_axis_name="s")


def _hist_body(dst_hbm, ones_hbm, zeros_hbm, degp_hbm, idx_v, ones_v, acc_sh):
    c = lax.axis_index("c")
    t = lax.axis_index("s")
    pltpu.sync_copy(zeros_hbm.at[pl.ds(t * 648, 648)],
                    acc_sh.at[pl.ds(t * 648, 648)])
    pltpu.sync_copy(ones_hbm, ones_v)
    pltpu.sync_copy(dst_hbm.at[t, pl.ds(c * 40, 40)], idx_v)
    plsc.subcore_barrier()

    def step(j, carry):
        pltpu.sync_copy(ones_v, acc_sh.at[idx_v.at[j]], add=True)
        return carry

    lax.fori_loop(0, 40, step, 0)
    plsc.subcore_barrier()
    pltpu.sync_copy(acc_sh.at[pl.ds(t * 640, 640)],
                    degp_hbm.at[c, pl.ds(t * 640, 640)])


_hist = pl.kernel(
    _hist_body,
    out_type=jax.ShapeDtypeStruct((2, NPAD, 128), jnp.float32),
    mesh=_mesh,
    scratch_types=[
        pltpu.VMEM((40, CHUNK), jnp.int32),
        pltpu.VMEM((CHUNK, 128), jnp.float32),
        pltpu.VMEM_SHARED((ACC_ROWS, 128), jnp.float32),
    ],
)


def _scat1_body(tbl_hbm, src_hbm, dst_hbm, zeros_hbm, out_hbm,
                idx_s, idx_d, rows_v, acc_sh, gsem):
    c = lax.axis_index("c")
    t = lax.axis_index("s")
    pltpu.sync_copy(zeros_hbm.at[pl.ds(t * 648, 648)],
                    acc_sh.at[pl.ds(t * 648, 648)])
    pltpu.sync_copy(src_hbm.at[c, t], idx_s)
    pltpu.sync_copy(dst_hbm.at[t], idx_d)
    plsc.subcore_barrier()

    def step(j, carry):
        pltpu.async_copy(tbl_hbm.at[idx_s.at[j]], rows_v, gsem).wait()
        pltpu.sync_copy(rows_v, acc_sh.at[idx_d.at[j]], add=True)
        return carry

    lax.fori_loop(0, NCHUNK, step, 0)
    plsc.subcore_barrier()
    pltpu.sync_copy(acc_sh.at[pl.ds(t * 640, 640)],
                    out_hbm.at[c, pl.ds(t * 640, 640)])


_scatter1 = pl.kernel(
    _scat1_body,
    out_type=jax.ShapeDtypeStruct((2, NPAD, 128), jnp.float32),
    mesh=_mesh,
    scratch_types=[
        pltpu.VMEM((NCHUNK, CHUNK), jnp.int32),
        pltpu.VMEM((NCHUNK, CHUNK), jnp.int32),
        pltpu.VMEM((CHUNK, 128), jnp.float32),
        pltpu.VMEM_SHARED((ACC_ROWS, 128), jnp.float32),
        pltpu.SemaphoreType.DMA,
    ],
)


def _scat2_body(tbl_hbm, src_hbm, dst_hbm, zeros_hbm, out_hbm,
                idx_s, idx_d, rows_v, acc_sh, gsem):
    c = lax.axis_index("c")
    t = lax.axis_index("s")
    pltpu.sync_copy(zeros_hbm.at[pl.ds(t * 648, 648)],
                    acc_sh.at[pl.ds(t * 648, 648)])
    pltpu.sync_copy(src_hbm.at[t, pl.ds(c * 40, 40)], idx_s)
    pltpu.sync_copy(dst_hbm.at[t, pl.ds(c * 40, 40)], idx_d)
    plsc.subcore_barrier()

    def step(j, carry):
        pltpu.async_copy(tbl_hbm.at[idx_s.at[j]], rows_v, gsem).wait()
        pltpu.sync_copy(rows_v, acc_sh.at[idx_d.at[j]], add=True)
        return carry

    lax.fori_loop(0, NCHUNK // 2, step, 0)
    plsc.subcore_barrier()
    pltpu.sync_copy(acc_sh.at[pl.ds(t * 640, 640)],
                    out_hbm.at[c, pl.ds(t * 640, 640)])


_scatter2 = pl.kernel(
    _scat2_body,
    out_type=jax.ShapeDtypeStruct((2, NPAD, 128), jnp.float32),
    mesh=_mesh,
    scratch_types=[
        pltpu.VMEM((NCHUNK // 2, CHUNK), jnp.int32),
        pltpu.VMEM((NCHUNK // 2, CHUNK), jnp.int32),
        pltpu.VMEM((CHUNK, 128), jnp.float32),
        pltpu.VMEM_SHARED((ACC_ROWS, 128), jnp.float32),
        pltpu.SemaphoreType.DMA,
    ],
)


def _dinv(dr):
    deg = dr[0, :, 0:1] + dr[1, :, 0:1] + 1.0
    return lax.rsqrt(jnp.maximum(deg, 1.0))


def _tc1_body(xr, dr, wr, ur):
    dinv = _dinv(dr)
    h = jnp.dot(xr[...], wr[...], preferred_element_type=jnp.float32)
    u = h * dinv
    ur[0] = u[:, :128]
    ur[1] = u[:, 128:]


def _tc2_body(sr, upr, dr, br, wr, ur):
    dinv = _dinv(dr)
    z = jnp.concatenate([sr[0] + upr[0], sr[1] + upr[1]], axis=1)
    z = jnp.maximum(z * dinv + br[...], 0.0)
    m = jnp.dot(z, wr[...], preferred_element_type=jnp.float32) * dinv
    rows = lax.broadcasted_iota(jnp.int32, (R_BLK, 1), 0) + pl.program_id(0) * R_BLK
    ur[...] = jnp.where(rows < N, m, 0.0)


def _tc3_body(sr, upr, dr, br, wr, bfr, outr):
    dinv = _dinv(dr)
    z = sr[0] + sr[1] + upr[...]
    z = jnp.maximum(z * dinv + br[...], 0.0)
    outr[...] = jnp.dot(z, wr[...], preferred_element_type=jnp.float32) + bfr[...]


def _row_spec(k):
    return pl.BlockSpec((2, R_BLK, k), lambda i: (0, i, 0))


def _flat_spec(k):
    return pl.BlockSpec((R_BLK, k), lambda i: (i, 0))


def _full_spec(shape):
    nd = len(shape)
    return pl.BlockSpec(shape, lambda i: (0,) * nd)


_GRID = NPAD // R_BLK

_tc1 = pl.pallas_call(
    _tc1_body,
    grid=(_GRID,),
    in_specs=[
        _flat_spec(D_IN),
        _row_spec(128),
        _full_spec((D_IN, D_H1)),
    ],
    out_specs=_row_spec(128),
    out_shape=jax.ShapeDtypeStruct((2, NPAD, 128), jnp.float32),
)

_tc2 = pl.pallas_call(
    _tc2_body,
    grid=(_GRID,),
    in_specs=[
        _row_spec(128),
        _row_spec(128),
        _row_spec(128),
        _full_spec((1, D_H1)),
        _full_spec((D_H1, D_H2)),
    ],
    out_specs=_flat_spec(D_H2),
    out_shape=jax.ShapeDtypeStruct((NPAD, D_H2), jnp.float32),
)

_tc3 = pl.pallas_call(
    _tc3_body,
    grid=(_GRID,),
    in_specs=[
        _row_spec(128),
        _flat_spec(D_H2),
        _row_spec(128),
        _full_spec((1, D_H2)),
        _full_spec((D_H2, D_OUT)),
        _full_spec((1, D_OUT)),
    ],
    out_specs=_flat_spec(D_OUT),
    out_shape=jax.ShapeDtypeStruct((NPAD, D_OUT), jnp.float32),
)


@jax.jit
def kernel(x, edge_index, W1, b1, W2, b2, Wfc, bfc):
    x_pad = jnp.zeros((NPAD, D_IN), jnp.float32).at[:N].set(x)
    src = edge_index[0]
    dst = edge_index[1]
    pad = EPAD - E
    src_p = jnp.concatenate([src, jnp.full((pad,), N, jnp.int32)])
    dst_p = jnp.concatenate([dst, jnp.full((pad,), DUMMY, jnp.int32)])
    src3 = src_p.reshape(16, NCHUNK, CHUNK)
    src_off = src3[None] + (jnp.arange(2, dtype=jnp.int32) * NPAD)[:, None, None, None]
    dst3 = dst_p.reshape(16, NCHUNK, CHUNK)

    ones128 = jnp.ones((CHUNK, 128), jnp.float32)
    zeros128 = jnp.zeros((ACC_ROWS, 128), jnp.float32)

    degp = _hist(dst3, ones128, zeros128)
    u1 = _tc1(x_pad, degp, W1)
    s1 = _scatter1(u1.reshape(2 * NPAD, 128), src_off, dst3, zeros128)
    u2 = _tc2(s1, u1, degp, b1.reshape(1, D_H1), W2)
    s2 = _scatter2(u2, src3, dst3, zeros128)
    out = _tc3(s2, u2, degp, b2.reshape(1, D_H2), Wfc, bfc.reshape(1, D_OUT))
    return out[:N]


# speedup vs baseline: 8.7331x; 1.3600x over previous
import jax
import jax.numpy as jnp
from jax import lax
from jax.experimental import pallas as pl
from jax.experimental.pallas import tpu as pltpu
from jax.experimental.pallas import tpu_sc as plsc

N = 10000
E = 160000
D_IN = 256
D_H1 = 256
D_H2 = 128
D_OUT = 64

NPAD = 10240
DUMMY = NPAD
ACC_ROWS = 10368
CHUNK = 128
NCHUNK = 80
EPAD = 16 * NCHUNK * CHUNK
R_BLK = 1024

_mesh = plsc.VectorSubcoreMesh(core_axis_name="c", subcore_axis_name="s")


def _hist_body(dst_hbm, ones_hbm, zeros_hbm, degp_hbm, idx_v, ones_v, acc_sh):
    c = lax.axis_index("c")
    t = lax.axis_index("s")
    pltpu.sync_copy(zeros_hbm.at[pl.ds(t * 648, 648)],
                    acc_sh.at[pl.ds(t * 648, 648)])
    pltpu.sync_copy(ones_hbm, ones_v)
    pltpu.sync_copy(dst_hbm.at[t, pl.ds(c * 40, 40)], idx_v)
    plsc.subcore_barrier()

    def step(j, carry):
        pltpu.sync_copy(ones_v, acc_sh.at[idx_v.at[j]], add=True)
        return carry

    lax.fori_loop(0, 40, step, 0)
    plsc.subcore_barrier()
    pltpu.sync_copy(acc_sh.at[pl.ds(t * 640, 640)],
                    degp_hbm.at[c, pl.ds(t * 640, 640)])


_hist = pl.kernel(
    _hist_body,
    out_type=jax.ShapeDtypeStruct((2, NPAD, 128), jnp.float32),
    mesh=_mesh,
    scratch_types=[
        pltpu.VMEM((40, CHUNK), jnp.int32),
        pltpu.VMEM((CHUNK, 128), jnp.float32),
        pltpu.VMEM_SHARED((ACC_ROWS, 128), jnp.float32),
    ],
)


def _make_scatter(edge_split):
    nh = 1 if edge_split else 2
    nc = 40
    ngrp = nc // 2

    def body(tbl_hbm, src_hbm, dst_hbm, zeros_hbm, out_hbm,
             idx_s, idx_d, b0, b1, acc_sh, g0, g1, s0, s1):
        c = lax.axis_index("c")
        t = lax.axis_index("s")
        bufs = (b0, b1)
        gsems = (g0, g1)
        ssems = (s0, s1)
        pltpu.sync_copy(zeros_hbm.at[pl.ds(t * 648, 648)],
                        acc_sh.at[pl.ds(t * 648, 648)])
        plsc.subcore_barrier()

        for h in range(nh):
            if edge_split:
                pltpu.sync_copy(src_hbm.at[t, pl.ds(c * nc, nc)], idx_s)
                pltpu.sync_copy(dst_hbm.at[t, pl.ds(c * nc, nc)], idx_d)
            else:
                pltpu.sync_copy(src_hbm.at[c, t, pl.ds(h * nc, nc)], idx_s)
                pltpu.sync_copy(dst_hbm.at[t, pl.ds(h * nc, nc)], idx_d)

            for k in range(2):
                pltpu.async_copy(tbl_hbm.at[idx_s.at[k]], bufs[k], gsems[k])

            def grp(g, carry):
                base = g * 2
                for k in range(2):
                    pltpu.make_async_copy(tbl_hbm.at[idx_s.at[base + k]],
                                          bufs[k], gsems[k]).wait()
                    pltpu.async_copy(bufs[k], acc_sh.at[idx_d.at[base + k]],
                                     ssems[k], add=True)

                @pl.when(g < ngrp - 1)
                def _():
                    for k in range(2):
                        pltpu.make_async_copy(bufs[k],
                                              acc_sh.at[idx_d.at[base + k]],
                                              ssems[k]).wait()
                        pltpu.async_copy(tbl_hbm.at[idx_s.at[base + 2 + k]],
                                         bufs[k], gsems[k])

                return carry

            lax.fori_loop(0, ngrp, grp, 0)
            last = (ngrp - 1) * 2
            for k in range(2):
                pltpu.make_async_copy(bufs[k], acc_sh.at[idx_d.at[last + k]],
                                      ssems[k]).wait()

        plsc.subcore_barrier()
        pltpu.sync_copy(acc_sh.at[pl.ds(t * 640, 640)],
                        out_hbm.at[c, pl.ds(t * 640, 640)])

    return pl.kernel(
        body,
        out_type=jax.ShapeDtypeStruct((2, NPAD, 128), jnp.float32),
        mesh=_mesh,
        scratch_types=[
            pltpu.VMEM((nc, CHUNK), jnp.int32),
            pltpu.VMEM((nc, CHUNK), jnp.int32),
            pltpu.VMEM((CHUNK, 128), jnp.float32),
            pltpu.VMEM((CHUNK, 128), jnp.float32),
            pltpu.VMEM_SHARED((ACC_ROWS, 128), jnp.float32),
            pltpu.SemaphoreType.DMA,
            pltpu.SemaphoreType.DMA,
            pltpu.SemaphoreType.DMA,
            pltpu.SemaphoreType.DMA,
        ],
    )


_scatter1 = _make_scatter(edge_split=False)
_scatter2 = _make_scatter(edge_split=True)


def _dinv(dr):
    deg = dr[0, :, 0:1] + dr[1, :, 0:1] + 1.0
    return lax.rsqrt(jnp.maximum(deg, 1.0))


def _tcv_body(xr, dr, vr):
    dinv = _dinv(dr)
    v = xr[...] * dinv
    vr[0] = v[:, :128]
    vr[1] = v[:, 128:]


def _tc2_body(sr, vr, dr, br, w1r, w2r, ur):
    dinv = _dinv(dr)
    p = jnp.concatenate([sr[0] + vr[0], sr[1] + vr[1]], axis=1)
    q = jnp.dot(p, w1r[...], preferred_element_type=jnp.float32) * dinv + br[...]
    z = jnp.maximum(q, 0.0)
    m = jnp.dot(z, w2r[...], preferred_element_type=jnp.float32) * dinv
    rows = lax.broadcasted_iota(jnp.int32, (R_BLK, 1), 0) + pl.program_id(0) * R_BLK
    ur[...] = jnp.where(rows < N, m, 0.0)


def _tc3_body(sr, upr, dr, br, wr, bfr, outr):
    dinv = _dinv(dr)
    z = sr[0] + sr[1] + upr[...]
    z = jnp.maximum(z * dinv + br[...], 0.0)
    outr[...] = jnp.dot(z, wr[...], preferred_element_type=jnp.float32) + bfr[...]


def _row_spec(k):
    return pl.BlockSpec((2, R_BLK, k), lambda i: (0, i, 0))


def _flat_spec(k):
    return pl.BlockSpec((R_BLK, k), lambda i: (i, 0))


def _full_spec(shape):
    nd = len(shape)
    return pl.BlockSpec(shape, lambda i: (0,) * nd)


_GRID = NPAD // R_BLK

_tcv = pl.pallas_call(
    _tcv_body,
    grid=(_GRID,),
    in_specs=[
        _flat_spec(D_IN),
        _row_spec(128),
    ],
    out_specs=_row_spec(128),
    out_shape=jax.ShapeDtypeStruct((2, NPAD, 128), jnp.float32),
)

_tc2 = pl.pallas_call(
    _tc2_body,
    grid=(_GRID,),
    in_specs=[
        _row_spec(128),
        _row_spec(128),
        _row_spec(128),
        _full_spec((1, D_H1)),
        _full_spec((D_IN, D_H1)),
        _full_spec((D_H1, D_H2)),
    ],
    out_specs=_flat_spec(D_H2),
    out_shape=jax.ShapeDtypeStruct((NPAD, D_H2), jnp.float32),
)

_tc3 = pl.pallas_call(
    _tc3_body,
    grid=(_GRID,),
    in_specs=[
        _row_spec(128),
        _flat_spec(D_H2),
        _row_spec(128),
        _full_spec((1, D_H2)),
        _full_spec((D_H2, D_OUT)),
        _full_spec((1, D_OUT)),
    ],
    out_specs=_flat_spec(D_OUT),
    out_shape=jax.ShapeDtypeStruct((NPAD, D_OUT), jnp.float32),
)


@jax.jit
def kernel(x, edge_index, W1, b1, W2, b2, Wfc, bfc):
    x_pad = jnp.zeros((NPAD, D_IN), jnp.float32).at[:N].set(x)
    src = edge_index[0]
    dst = edge_index[1]
    pad = EPAD - E
    src_p = jnp.concatenate([src, jnp.full((pad,), N, jnp.int32)])
    dst_p = jnp.concatenate([dst, jnp.full((pad,), DUMMY, jnp.int32)])
    src3 = src_p.reshape(16, NCHUNK, CHUNK)
    src_off = src3[None] + (jnp.arange(2, dtype=jnp.int32) * NPAD)[:, None, None, None]
    dst3 = dst_p.reshape(16, NCHUNK, CHUNK)

    ones128 = jnp.ones((CHUNK, 128), jnp.float32)
    zeros128 = jnp.zeros((ACC_ROWS, 128), jnp.float32)

    src3i = src3.reshape(16, NCHUNK // 2, 2, CHUNK).transpose(0, 2, 1, 3).reshape(16, NCHUNK, CHUNK)
    dst3i = dst3.reshape(16, NCHUNK // 2, 2, CHUNK).transpose(0, 2, 1, 3).reshape(16, NCHUNK, CHUNK)

    degp = _hist(dst3, ones128, zeros128)
    v = _tcv(x_pad, degp)
    s1 = _scatter1(v.reshape(2 * NPAD, 128), src_off, dst3, zeros128)
    u2 = _tc2(s1, v, degp, b1.reshape(1, D_H1), W1, W2)
    s2 = _scatter2(u2, src3i, dst3i, zeros128)
    out = _tc3(s2, u2, degp, b2.reshape(1, D_H2), Wfc, bfc.reshape(1, D_OUT))
    return out[:N]


# speedup vs baseline: 8.7695x; 1.0042x over previous
import jax
import jax.numpy as jnp
from jax import lax
from jax.experimental import pallas as pl
from jax.experimental.pallas import tpu as pltpu
from jax.experimental.pallas import tpu_sc as plsc

N = 10000
E = 160000
D_IN = 256
D_H1 = 256
D_H2 = 128
D_OUT = 64

NPAD = 10240
DUMMY = NPAD
ACC_ROWS = 10368
CHUNK = 128
NCHUNK = 80
EPAD = 16 * NCHUNK * CHUNK
R_BLK = 1024

_mesh = plsc.VectorSubcoreMesh(core_axis_name="c", subcore_axis_name="s")


def _hist_body(dst_hbm, ones_hbm, zeros_hbm, degp_hbm, idx_v, ones_v, acc_sh,
               hsem):
    c = lax.axis_index("c")
    t = lax.axis_index("s")
    pltpu.sync_copy(zeros_hbm.at[pl.ds(t * 648, 648)],
                    acc_sh.at[pl.ds(t * 648, 648)])
    pltpu.sync_copy(ones_hbm, ones_v)
    pltpu.sync_copy(dst_hbm.at[t, pl.ds(c * 40, 40)], idx_v)
    plsc.subcore_barrier()

    def step(j, carry):
        pltpu.async_copy(ones_v, acc_sh.at[idx_v.at[j]], hsem, add=True)
        return carry

    lax.fori_loop(0, 40, step, 0)

    def drain(j, carry):
        pltpu.make_async_copy(ones_v, acc_sh.at[idx_v.at[j]], hsem).wait()
        return carry

    lax.fori_loop(0, 40, drain, 0)
    plsc.subcore_barrier()
    pltpu.sync_copy(acc_sh.at[pl.ds(t * 640, 640)],
                    degp_hbm.at[c, pl.ds(t * 640, 640)])


_hist = pl.kernel(
    _hist_body,
    out_type=jax.ShapeDtypeStruct((2, NPAD, 128), jnp.float32),
    mesh=_mesh,
    scratch_types=[
        pltpu.VMEM((40, CHUNK), jnp.int32),
        pltpu.VMEM((CHUNK, 128), jnp.float32),
        pltpu.VMEM_SHARED((ACC_ROWS, 128), jnp.float32),
        pltpu.SemaphoreType.DMA,
    ],
)


def _make_scatter(edge_split, chunk, nbuf):
    nh = 2
    per_tile = (EPAD // 32) if edge_split else (EPAD // 16)
    nc = per_tile // nh // chunk
    del edge_split
    ngrp = nc // nbuf

    def body(tbl_hbm, src_hbm, dst_hbm, zeros_hbm, out_hbm, *scr):
        idx_s, idx_d = scr[0], scr[1]
        bufs = scr[2:2 + nbuf]
        acc_sh = scr[2 + nbuf]
        gsems = scr[3 + nbuf:3 + 2 * nbuf]
        ssems = scr[3 + 2 * nbuf:3 + 3 * nbuf]
        c = lax.axis_index("c")
        t = lax.axis_index("s")
        zcp = pltpu.async_copy(zeros_hbm.at[pl.ds(t * 648, 648)],
                               acc_sh.at[pl.ds(t * 648, 648)], gsems[0])
        first = True

        for h in range(nh):
            scp = pltpu.async_copy(src_hbm.at[c, t, pl.ds(h * nc, nc)],
                                   idx_s, ssems[0])
            dcp = pltpu.async_copy(dst_hbm.at[c, t, pl.ds(h * nc, nc)],
                                   idx_d, ssems[1])
            scp.wait()
            dcp.wait()
            if first:
                zcp.wait()
                plsc.subcore_barrier()
                first = False

            for k in range(nbuf):
                pltpu.async_copy(tbl_hbm.at[idx_s.at[k]], bufs[k], gsems[k])

            def grp(g, carry):
                base = g * nbuf
                for k in range(nbuf):
                    pltpu.make_async_copy(tbl_hbm.at[idx_s.at[base + k]],
                                          bufs[k], gsems[k]).wait()
                    pltpu.async_copy(bufs[k], acc_sh.at[idx_d.at[base + k]],
                                     ssems[k], add=True)

                @pl.when(g < ngrp - 1)
                def _():
                    for k in range(nbuf):
                        pltpu.make_async_copy(bufs[k],
                                              acc_sh.at[idx_d.at[base + k]],
                                              ssems[k]).wait()
                        pltpu.async_copy(tbl_hbm.at[idx_s.at[base + nbuf + k]],
                                         bufs[k], gsems[k])

                return carry

            lax.fori_loop(0, ngrp, grp, 0)
            last = (ngrp - 1) * nbuf
            for k in range(nbuf):
                pltpu.make_async_copy(bufs[k], acc_sh.at[idx_d.at[last + k]],
                                      ssems[k]).wait()

        plsc.subcore_barrier()
        pltpu.sync_copy(acc_sh.at[pl.ds(t * 640, 640)],
                        out_hbm.at[c, pl.ds(t * 640, 640)])

    return pl.kernel(
        body,
        out_type=jax.ShapeDtypeStruct((2, NPAD, 128), jnp.float32),
        mesh=_mesh,
        scratch_types=(
            [pltpu.VMEM((nc, chunk), jnp.int32),
             pltpu.VMEM((nc, chunk), jnp.int32)]
            + [pltpu.VMEM((chunk, 128), jnp.float32) for _ in range(nbuf)]
            + [pltpu.VMEM_SHARED((ACC_ROWS, 128), jnp.float32)]
            + [pltpu.SemaphoreType.DMA for _ in range(2 * nbuf)]
        ),
    )


_scatter1 = _make_scatter(edge_split=False, chunk=128, nbuf=2)
_scatter2 = _make_scatter(edge_split=True, chunk=64, nbuf=4)


def _dinv(dr):
    deg = dr[0, :, 0:1] + dr[1, :, 0:1] + 1.0
    return lax.rsqrt(jnp.maximum(deg, 1.0))


def _tcv_body(xr, dr, vr, dvr):
    dinv = _dinv(dr)
    v = xr[...] * dinv
    vr[0] = v[:, :128]
    vr[1] = v[:, 128:]
    dvr[...] = dinv.reshape(1, R_BLK)


def _tc2_body(sr, vr, dr, br, w1r, w2r, ur):
    dinv = dr[...].reshape(R_BLK, 1)
    p = jnp.concatenate([sr[0] + vr[0], sr[1] + vr[1]], axis=1)
    q = jnp.dot(p, w1r[...], preferred_element_type=jnp.float32) * dinv + br[...]
    z = jnp.maximum(q, 0.0)
    m = jnp.dot(z, w2r[...], preferred_element_type=jnp.float32) * dinv
    rows = lax.broadcasted_iota(jnp.int32, (R_BLK, 1), 0) + pl.program_id(0) * R_BLK
    m = jnp.where(rows < N, m, 0.0)
    ur[0] = m
    ur[1] = m


def _tc3_body(sr, upr, dr, br, wr, bfr, outr):
    dinv = dr[...].reshape(R_BLK, 1)
    z = sr[0] + sr[1] + upr[0]
    z = jnp.maximum(z * dinv + br[...], 0.0)
    outr[...] = jnp.dot(z, wr[...], preferred_element_type=jnp.float32) + bfr[...]


def _row_spec(k):
    return pl.BlockSpec((2, R_BLK, k), lambda i: (0, i, 0))


def _flat_spec(k):
    return pl.BlockSpec((R_BLK, k), lambda i: (i, 0))


def _full_spec(shape):
    nd = len(shape)
    return pl.BlockSpec(shape, lambda i: (0,) * nd)


_GRID = NPAD // R_BLK

_tcv = pl.pallas_call(
    _tcv_body,
    grid=(_GRID,),
    in_specs=[
        _flat_spec(D_IN),
        _row_spec(128),
    ],
    out_specs=[
        _row_spec(128),
        pl.BlockSpec((1, R_BLK), lambda i: (0, i)),
    ],
    out_shape=[
        jax.ShapeDtypeStruct((2, NPAD, 128), jnp.float32),
        jax.ShapeDtypeStruct((1, NPAD), jnp.float32),
    ],
)

_tc2 = pl.pallas_call(
    _tc2_body,
    grid=(_GRID,),
    in_specs=[
        _row_spec(128),
        _row_spec(128),
        pl.BlockSpec((1, R_BLK), lambda i: (0, i)),
        _full_spec((1, D_H1)),
        _full_spec((D_IN, D_H1)),
        _full_spec((D_H1, D_H2)),
    ],
    out_specs=_row_spec(128),
    out_shape=jax.ShapeDtypeStruct((2, NPAD, D_H2), jnp.float32),
)

_tc3 = pl.pallas_call(
    _tc3_body,
    grid=(_GRID,),
    in_specs=[
        _row_spec(128),
        _row_spec(128),
        pl.BlockSpec((1, R_BLK), lambda i: (0, i)),
        _full_spec((1, D_H2)),
        _full_spec((D_H2, D_OUT)),
        _full_spec((1, D_OUT)),
    ],
    out_specs=_flat_spec(D_OUT),
    out_shape=jax.ShapeDtypeStruct((NPAD, D_OUT), jnp.float32),
)


@jax.jit
def kernel(x, edge_index, W1, b1, W2, b2, Wfc, bfc):
    x_pad = jnp.zeros((NPAD, D_IN), jnp.float32).at[:N].set(x)
    src = edge_index[0]
    dst = edge_index[1]
    pad = EPAD - E
    src_p = jnp.concatenate([src, jnp.full((pad,), N, jnp.int32)])
    dst_p = jnp.concatenate([dst, jnp.full((pad,), DUMMY, jnp.int32)])
    src3 = src_p.reshape(16, NCHUNK, CHUNK)
    src_off = src3[None] + (jnp.arange(2, dtype=jnp.int32) * NPAD)[:, None, None, None]
    dst3 = dst_p.reshape(16, NCHUNK, CHUNK)

    ones128 = jnp.ones((CHUNK, 128), jnp.float32)
    zeros128 = jnp.zeros((ACC_ROWS, 128), jnp.float32)

    dst4_1 = jnp.stack([dst3, dst3])
    src64 = src_p.reshape(16, 160, 64)
    dst64 = dst_p.reshape(16, 160, 64)
    off = (jnp.arange(2, dtype=jnp.int32) * NPAD)[:, None, None, None]
    src4_2 = jnp.stack([src64[:, 0::2], src64[:, 1::2]]) + off
    dst4_2 = jnp.stack([dst64[:, 0::2], dst64[:, 1::2]])

    degp = _hist(dst3, ones128, zeros128)
    v, dinv = _tcv(x_pad, degp)
    s1 = _scatter1(v.reshape(2 * NPAD, 128), src_off, dst4_1, zeros128)
    u2 = _tc2(s1, v, dinv, b1.reshape(1, D_H1), W1, W2)
    s2 = _scatter2(u2.reshape(2 * NPAD, 128), src4_2, dst4_2, zeros128)
    out = _tc3(s2, u2, dinv, b2.reshape(1, D_H2), Wfc, bfc.reshape(1, D_OUT))
    return out[:N]
